# Initial kernel scaffold; baseline (speedup 1.0000x reference)
import jax
import jax.numpy as jnp
from jax.experimental import pallas as pl


def kernel(shake_audio, shake_acc):
    raise NotImplementedError("write your pallas kernel here")


# speedup vs baseline: 2.4488x; 2.4488x over previous
import functools

import numpy as np
import jax
import jax.numpy as jnp
from jax import lax
from jax.experimental import pallas as pl
from jax.experimental.pallas import tpu as pltpu
from jax.experimental.pallas import tpu_sc as plsc

P = 0.5

B, C, T, D = 64, 3, 128, 256
ROWS = B * C * T
NC, NS = 2, 16
NW = NC * NS
RPW = ROWS // NW
CH = T
NCHUNK = RPW // CH
NSTREAM = 2 * NCHUNK

_CONSTS = None


def _get_consts():
    global _CONSTS
    if _CONSTS is None:
        with jax.ensure_compile_time_eval():
            key = jax.random.key(42)
            coins, perms = [], []
            for i in range(2):
                kk = jax.random.fold_in(key, i)
                kc, kp = jax.random.split(kk)
                coins.append(bool(jax.random.uniform(kc) < P))
                perms.append(np.asarray(jax.random.permutation(kp, T)))
        eff = [p if c else np.arange(T, dtype=np.int32) for c, p in zip(coins, perms)]
        idx = np.zeros((NW, NSTREAM, CH), dtype=np.int32)
        for w in range(NW):
            for j in range(NCHUNK):
                base = (w * NCHUNK + j) * CH
                idx[w, j, :] = base + eff[0]
                idx[w, NCHUNK + j, :] = base + eff[1]
        labels = np.tile(
            np.array([[float(coins[0]), float(coins[1])]], dtype=np.float32),
            (B, 1))
        _CONSTS = (idx, labels)
    return _CONSTS


@functools.partial(
    pl.kernel,
    out_type=(
        jax.ShapeDtypeStruct((ROWS, D), jnp.float32),
        jax.ShapeDtypeStruct((ROWS, D), jnp.float32),
    ),
    mesh=plsc.VectorSubcoreMesh(core_axis_name="c", subcore_axis_name="s"),
    scratch_types=[
        pltpu.VMEM((NSTREAM, CH), jnp.int32),
        pltpu.VMEM((CH, D), jnp.float32),
        pltpu.VMEM((CH, D), jnp.float32),
        pltpu.SemaphoreType.DMA,
        pltpu.SemaphoreType.DMA,
    ],
)
def _permute_rows(audio_hbm, acc_hbm, idx_hbm,
                  audio_out, acc_out,
                  idx_v, buf0, buf1, sem0, sem1):
    wid = lax.axis_index("s") * NC + lax.axis_index("c")
    pltpu.sync_copy(idx_hbm.at[wid], idx_v)

    srcs = (audio_hbm, acc_hbm)
    dsts = (audio_out, acc_out)
    bufs = (buf0, buf1)
    sems = (sem0, sem1)

    def start(i):
        t, j = divmod(i, NCHUNK)
        return pltpu.async_copy(srcs[t].at[idx_v.at[i]], bufs[i % 2], sems[i % 2])

    def drain(i, handle):
        t, j = divmod(i, NCHUNK)
        handle.wait()
        base = (wid * NCHUNK + j) * CH
        pltpu.sync_copy(bufs[i % 2], dsts[t].at[pl.ds(base, CH)])

    handle = start(0)
    for i in range(1, NSTREAM):
        nxt = start(i)
        drain(i - 1, handle)
        handle = nxt
    drain(NSTREAM - 1, handle)


def kernel(shake_audio, shake_acc):
    idx_np, labels_np = _get_consts()
    a2 = shake_audio.reshape(ROWS, D)
    c2 = shake_acc.reshape(ROWS, D)
    out_a, out_c = _permute_rows(a2, c2, jnp.asarray(idx_np))
    return (out_a.reshape(shake_audio.shape),
            out_c.reshape(shake_acc.shape),
            jnp.asarray(labels_np))


# speedup vs baseline: 2.4517x; 1.0012x over previous
import functools

import numpy as np
import jax
import jax.numpy as jnp
from jax import lax
from jax.experimental import pallas as pl
from jax.experimental.pallas import tpu as pltpu
from jax.experimental.pallas import tpu_sc as plsc

P = 0.5

B, C, T, D = 64, 3, 128, 256
ROWS = B * C * T
NC, NS = 2, 16
NW = NC * NS
RPW = ROWS // NW
CH = 64
NCHUNK = RPW // CH
NSTREAM = 2 * NCHUNK
NBUF = 4
LAG = 2

_CONSTS = None


def _get_consts():
    global _CONSTS
    if _CONSTS is None:
        with jax.ensure_compile_time_eval():
            key = jax.random.key(42)
            coins, perms = [], []
            for i in range(2):
                kk = jax.random.fold_in(key, i)
                kc, kp = jax.random.split(kk)
                coins.append(bool(jax.random.uniform(kc) < P))
                perms.append(np.asarray(jax.random.permutation(kp, T)))
        eff = [p if c else np.arange(T, dtype=np.int32) for c, p in zip(coins, perms)]
        r = np.arange(ROWS, dtype=np.int64)
        idx = np.zeros((NW, NSTREAM, CH), dtype=np.int32)
        for t in range(2):
            src = (r // T) * T + eff[t][r % T]
            idx[:, t * NCHUNK:(t + 1) * NCHUNK, :] = src.reshape(NW, NCHUNK, CH)
        labels = np.tile(
            np.array([[float(coins[0]), float(coins[1])]], dtype=np.float32),
            (B, 1))
        _CONSTS = (idx, labels)
    return _CONSTS


@functools.partial(
    pl.kernel,
    out_type=(
        jax.ShapeDtypeStruct((ROWS, D), jnp.float32),
        jax.ShapeDtypeStruct((ROWS, D), jnp.float32),
    ),
    mesh=plsc.VectorSubcoreMesh(core_axis_name="c", subcore_axis_name="s"),
    scratch_types=[
        pltpu.VMEM((NSTREAM, CH), jnp.int32),
        [pltpu.VMEM((CH, D), jnp.float32) for _ in range(NBUF)],
        [pltpu.SemaphoreType.DMA for _ in range(NBUF)],
        [pltpu.SemaphoreType.DMA for _ in range(NBUF)],
    ],
)
def _permute_rows(audio_hbm, acc_hbm, idx_hbm,
                  audio_out, acc_out,
                  idx_v, bufs, in_sems, out_sems):
    wid = lax.axis_index("s") * NC + lax.axis_index("c")
    pltpu.sync_copy(idx_hbm.at[wid], idx_v)

    srcs = (audio_hbm, acc_hbm)
    dsts = (audio_out, acc_out)

    in_h = [None] * NBUF
    out_h = [None] * NBUF
    for i in range(NSTREAM + LAG):
        if i < NSTREAM:
            b = i % NBUF
            t = i // NCHUNK
            if out_h[b] is not None:
                out_h[b].wait()
            in_h[b] = pltpu.async_copy(
                srcs[t].at[idx_v.at[i]], bufs[b], in_sems[b])
        j = i - LAG
        if 0 <= j:
            bj = j % NBUF
            tj, cj = divmod(j, NCHUNK)
            in_h[bj].wait()
            base = wid * RPW + cj * CH
            out_h[bj] = pltpu.async_copy(
                bufs[bj], dsts[tj].at[pl.ds(base, CH)], out_sems[bj])
    for b in range(NBUF):
        if out_h[b] is not None:
            out_h[b].wait()


def kernel(shake_audio, shake_acc):
    idx_np, labels_np = _get_consts()
    a2 = shake_audio.reshape(ROWS, D)
    c2 = shake_acc.reshape(ROWS, D)
    out_a, out_c = _permute_rows(a2, c2, jnp.asarray(idx_np))
    return (out_a.reshape(shake_audio.shape),
            out_c.reshape(shake_acc.shape),
            jnp.asarray(labels_np))


# speedup vs baseline: 2.5017x; 1.0204x over previous
import functools

import numpy as np
import jax
import jax.numpy as jnp
from jax import lax
from jax.experimental import pallas as pl
from jax.experimental.pallas import tpu as pltpu
from jax.experimental.pallas import tpu_sc as plsc

P = 0.5

B, C, T, D = 64, 3, 128, 256
ROWS = B * C * T
NC, NS = 2, 16
NW = NC * NS
RPW = ROWS // NW
CH = 64
NCHUNK = RPW // CH
NSTREAM = 2 * NCHUNK
NBUF = 6
LAG = 3

_CONSTS = None


def _get_consts():
    global _CONSTS
    if _CONSTS is None:
        with jax.ensure_compile_time_eval():
            key = jax.random.key(42)
            coins, perms = [], []
            for i in range(2):
                kk = jax.random.fold_in(key, i)
                kc, kp = jax.random.split(kk)
                coins.append(bool(jax.random.uniform(kc) < P))
                perms.append(np.asarray(jax.random.permutation(kp, T)))
        eff = [p if c else np.arange(T, dtype=np.int32) for c, p in zip(coins, perms)]
        r = np.arange(ROWS, dtype=np.int64)
        idx = np.zeros((NW, NSTREAM, CH), dtype=np.int32)
        for t in range(2):
            src = (r // T) * T + eff[t][r % T]
            idx[:, t * NCHUNK:(t + 1) * NCHUNK, :] = src.reshape(NW, NCHUNK, CH)
        labels = np.tile(
            np.array([[float(coins[0]), float(coins[1])]], dtype=np.float32),
            (B, 1))
        _CONSTS = (idx, labels)
    return _CONSTS


@functools.partial(
    pl.kernel,
    out_type=(
        jax.ShapeDtypeStruct((ROWS, D), jnp.float32),
        jax.ShapeDtypeStruct((ROWS, D), jnp.float32),
    ),
    mesh=plsc.VectorSubcoreMesh(core_axis_name="c", subcore_axis_name="s"),
    scratch_types=[
        pltpu.VMEM((NSTREAM, CH), jnp.int32),
        [pltpu.VMEM((CH, D), jnp.float32) for _ in range(NBUF)],
        [pltpu.SemaphoreType.DMA for _ in range(NBUF)],
        [pltpu.SemaphoreType.DMA for _ in range(NBUF)],
    ],
)
def _permute_rows(audio_hbm, acc_hbm, idx_hbm,
                  audio_out, acc_out,
                  idx_v, bufs, in_sems, out_sems):
    wid = lax.axis_index("s") * NC + lax.axis_index("c")
    pltpu.sync_copy(idx_hbm.at[wid], idx_v)

    srcs = (audio_hbm, acc_hbm)
    dsts = (audio_out, acc_out)

    in_h = [None] * NBUF
    out_h = [None] * NBUF
    for i in range(NSTREAM + LAG):
        if i < NSTREAM:
            b = i % NBUF
            t = i // NCHUNK
            if out_h[b] is not None:
                out_h[b].wait()
            in_h[b] = pltpu.async_copy(
                srcs[t].at[idx_v.at[i]], bufs[b], in_sems[b])
        j = i - LAG
        if 0 <= j:
            bj = j % NBUF
            tj, cj = divmod(j, NCHUNK)
            in_h[bj].wait()
            base = wid * RPW + cj * CH
            out_h[bj] = pltpu.async_copy(
                bufs[bj], dsts[tj].at[pl.ds(base, CH)], out_sems[bj])
    for b in range(NBUF):
        if out_h[b] is not None:
            out_h[b].wait()


def kernel(shake_audio, shake_acc):
    idx_np, labels_np = _get_consts()
    a2 = shake_audio.reshape(ROWS, D)
    c2 = shake_acc.reshape(ROWS, D)
    out_a, out_c = _permute_rows(a2, c2, jnp.asarray(idx_np))
    return (out_a.reshape(shake_audio.shape),
            out_c.reshape(shake_acc.shape),
            jnp.asarray(labels_np))
